# baseline (speedup 1.0000x reference)
import functools

import jax
import jax.numpy as jnp
from jax import lax
from jax.experimental import pallas as pl
from jax.experimental.pallas import tpu as pltpu
from jax.experimental.pallas import tpu_sc as plsc

Q = 1024
D = 128
M = 100000
K = 32
BM = 2048
R = BM // 128
QT = 512
QA = 256
NMB = (M + BM - 1) // BM

NEG = float("-inf")
POS = float("inf")


def _topk_body(q_ref, kv_ref, idx_ref, run_v, run_i, sv,
               L1, L2, A1, A2, LP, LR, done):
    mi = pl.program_id(1)

    @pl.when(mi == 0)
    def _init():
        run_v[:] = jnp.full((QT, K), NEG, jnp.float32)
        run_i[:] = lax.broadcasted_iota(jnp.int32, (QT, K), 1)

    q = q_ref[:]
    qn = q / (jnp.sqrt(jnp.sum(q * q, axis=-1, keepdims=True)) + 1e-12)
    kb = kv_ref[:, 0, :]
    dots2 = lax.dot_general(qn + qn, kb, (((1,), (1,)), ((), ())),
                            preferred_element_type=jnp.float32)
    ksq = jnp.sum(kb * kb, axis=-1)
    negs = dots2 - ksq[None, :]
    sv[:] = negs

    @pl.when(mi == NMB - 1)
    def _mask_tail():
        tail = lax.broadcasted_iota(jnp.int32, (QT, BM), 1)
        sv[:] = jnp.where(tail >= M - (NMB - 1) * BM, NEG, sv[:])

    m1 = jnp.full((QT, 128), NEG, jnp.float32)
    m2 = jnp.full((QT, 128), NEG, jnp.float32)
    a1 = jnp.full((QT, 128), -1, jnp.int32)
    a2 = jnp.full((QT, 128), -1, jnp.int32)
    for r in range(R):
        v = sv[:, r * 128:(r + 1) * 128]
        c1 = v > m1
        c2 = jnp.logical_not(c1) & (v > m2)
        m2 = jnp.where(c1, m1, jnp.where(c2, v, m2))
        a2 = jnp.where(c1, a1, jnp.where(c2, r, a2))
        m1 = jnp.where(c1, v, m1)
        a1 = jnp.where(c1, r, a1)
    L1[:] = m1
    L2[:] = m2
    A1[:] = a1
    A2[:] = a2
    LP[:] = jnp.full((QT, 128), POS, jnp.float32)
    LR[:] = jnp.full((QT, 128), -1, jnp.int32)
    done[0] = 0

    il = lax.broadcasted_iota(jnp.int32, (QT, 128), 1)

    def body(_, carry):
        @pl.when(done[0] == 0)
        def _round():
            l1 = L1[:]
            maxv = jnp.max(l1, axis=-1, keepdims=True)
            rv = run_v[:]
            rmin = jnp.min(rv, axis=-1, keepdims=True)
            imp = maxv > rmin
            nimp = jnp.max(imp.astype(jnp.int32))
            done[0] = 1 - nimp

            @pl.when(nimp == 1)
            def _insert():
                a1v = A1[:]
                pos = a1v * 128 + il
                p = jnp.min(jnp.where(l1 == maxv, pos, BM), axis=-1,
                            keepdims=True)
                wl = jnp.bitwise_and(p, 127)
                arow = jnp.right_shift(p, 7)
                upd = (il == wl) & imp
                l2 = L2[:]
                need = upd & (l2 == NEG)
                anyneed = jnp.max(need.astype(jnp.int32))
                L1[:] = jnp.where(upd, l2, l1)
                A1[:] = jnp.where(upd, A2[:], a1v)
                L2[:] = jnp.where(upd, NEG, l2)
                A2[:] = jnp.where(upd, -1, A2[:])
                LP[:] = jnp.where(upd, maxv, LP[:])
                LR[:] = jnp.where(upd, arow, LR[:])
                ri = run_i[:]
                ismin = rv == rmin
                mri = jnp.max(jnp.where(ismin, ri, -1), axis=-1,
                              keepdims=True)
                repl = ismin & (ri == mri) & imp
                run_v[:] = jnp.where(repl, maxv, rv)
                run_i[:] = jnp.where(repl, mi * BM + p, ri)

                @pl.when(anyneed == 1)
                def _refill():
                    lp = LP[:]
                    lr = LR[:]
                    n1 = jnp.full((QT, 128), NEG, jnp.float32)
                    n2 = jnp.full((QT, 128), NEG, jnp.float32)
                    b1 = jnp.full((QT, 128), -1, jnp.int32)
                    b2 = jnp.full((QT, 128), -1, jnp.int32)
                    for rr in range(R):
                        v = sv[:, rr * 128:(rr + 1) * 128]
                        ex = (v > lp) | ((v == lp) & (lr >= rr))
                        v = jnp.where(ex, NEG, v)
                        c1 = v > n1
                        c2 = jnp.logical_not(c1) & (v > n2)
                        n2 = jnp.where(c1, n1, jnp.where(c2, v, n2))
                        b2 = jnp.where(c1, b1, jnp.where(c2, rr, b2))
                        n1 = jnp.where(c1, v, n1)
                        b1 = jnp.where(c1, rr, b1)
                    sel = need
                    L1[:] = jnp.where(sel, n1, L1[:])
                    A1[:] = jnp.where(sel, b1, A1[:])
                    L2[:] = jnp.where(sel, n2, L2[:])
                    A2[:] = jnp.where(sel, b2, A2[:])

        return carry

    lax.fori_loop(0, K, body, 0, unroll=False)

    @pl.when(mi == NMB - 1)
    def _out():
        idx_ref[:] = run_i[:]


def _topk_call(queries, memory_kv):
    return pl.pallas_call(
        _topk_body,
        grid=(Q // QT, NMB),
        in_specs=[
            pl.BlockSpec((QT, D), lambda qi, mi: (qi, 0)),
            pl.BlockSpec((BM, 2, D), lambda qi, mi: (mi, 0, 0)),
        ],
        out_specs=pl.BlockSpec((QT, K), lambda qi, mi: (qi, 0)),
        out_shape=jax.ShapeDtypeStruct((Q, K), jnp.int32),
        scratch_shapes=[
            pltpu.VMEM((QT, K), jnp.float32),
            pltpu.VMEM((QT, K), jnp.int32),
            pltpu.VMEM((QT, BM), jnp.float32),
            pltpu.VMEM((QT, 128), jnp.float32),
            pltpu.VMEM((QT, 128), jnp.float32),
            pltpu.VMEM((QT, 128), jnp.int32),
            pltpu.VMEM((QT, 128), jnp.int32),
            pltpu.VMEM((QT, 128), jnp.float32),
            pltpu.VMEM((QT, 128), jnp.int32),
            pltpu.SMEM((1,), jnp.int32),
        ],
    )(queries, memory_kv)


NW = 32
ROWS = Q * K
RPW = ROWS // NW
CH = 128
NCH = RPW // CH


def _gather_call(table, idx):
    mesh = plsc.VectorSubcoreMesh(core_axis_name="c", subcore_axis_name="s")

    @functools.partial(
        pl.kernel,
        mesh=mesh,
        out_type=jax.ShapeDtypeStruct((ROWS, 2 * D), jnp.float32),
        scratch_types=[
            pltpu.VMEM((CH,), jnp.int32),
            pltpu.VMEM((CH, 2 * D), jnp.float32),
            pltpu.SemaphoreType.DMA,
        ],
    )
    def k(table_hbm, idx_hbm, out_hbm, idx_v, rows_v, sem):
        wid = lax.axis_index("s") * 2 + lax.axis_index("c")
        for c in range(NCH):
            base = wid * RPW + c * CH
            pltpu.sync_copy(idx_hbm.at[pl.ds(base, CH)], idx_v)
            pltpu.async_copy(table_hbm.at[idx_v], rows_v, sem).wait()
            pltpu.sync_copy(rows_v, out_hbm.at[pl.ds(base, CH)])

    return k(table, idx)


def _attn_body(topk_ref, q_ref, kv_ref, out_ref):
    q = q_ref[:]
    qn = q / (jnp.sqrt(jnp.sum(q * q, axis=-1, keepdims=True)) + 1e-12)
    rk = kv_ref[:, :, :D]
    rv = kv_ref[:, :, D:]
    scale = 1.0 / jnp.sqrt(jnp.float32(D))
    logits = jnp.sum(qn[:, None, :] * rk, axis=-1) * scale
    valid = lax.broadcasted_iota(jnp.int32, (QA, K), 1) < topk_ref[0]
    logits = jnp.where(valid, logits, NEG)
    m = jnp.max(logits, axis=-1, keepdims=True)
    e = jnp.exp(logits - m)
    w = e / jnp.sum(e, axis=-1, keepdims=True)
    out_ref[:] = jnp.sum(w[:, :, None] * rv, axis=1)


def _attn_call(topk, queries, kv):
    return pl.pallas_call(
        _attn_body,
        grid=(Q // QA,),
        in_specs=[
            pl.BlockSpec(memory_space=pltpu.SMEM),
            pl.BlockSpec((QA, D), lambda qi: (qi, 0)),
            pl.BlockSpec((QA, K, 2 * D), lambda qi: (qi, 0, 0)),
        ],
        out_specs=pl.BlockSpec((QA, D), lambda qi: (qi, 0)),
        out_shape=jax.ShapeDtypeStruct((Q, D), jnp.float32),
    )(topk, queries, kv)


def kernel(queries, memory_kv, topk):
    idx = _topk_call(queries, memory_kv)
    table = memory_kv.reshape(M, 2 * D)
    kv = _gather_call(table, idx.reshape(ROWS)).reshape(Q, K, 2 * D)
    topk_arr = jnp.asarray(topk, jnp.int32).reshape(1)
    return _attn_call(topk_arr, queries, kv)


# speedup vs baseline: 1.2203x; 1.2203x over previous
import functools

import jax
import jax.numpy as jnp
from jax import lax
from jax.experimental import pallas as pl
from jax.experimental.pallas import tpu as pltpu
from jax.experimental.pallas import tpu_sc as plsc

Q = 1024
D = 128
M = 100000
K = 32
BM = 512
QT = 512
QA = 256
NMB = (M + BM - 1) // BM
W = K + BM

NEG = float("-inf")


def _topk_body(q_ref, kv_ref, idx_ref, run_vs, run_is, sv, done):
    mi = pl.program_id(0)
    qi = pl.program_id(1)
    run_v = run_vs.at[qi]
    run_i = run_is.at[qi]

    @pl.when(mi == 0)
    def _init():
        run_v[:] = jnp.full((QT, K), NEG, jnp.float32)
        run_i[:] = lax.broadcasted_iota(jnp.int32, (QT, K), 1)

    q = q_ref[:]
    qn = q / (jnp.sqrt(jnp.sum(q * q, axis=-1, keepdims=True)) + 1e-12)
    kb = kv_ref[:, 0, :]
    dots2 = lax.dot_general(qn + qn, kb, (((1,), (1,)), ((), ())),
                            preferred_element_type=jnp.float32)
    ksq = jnp.sum(kb * kb, axis=-1)
    negs = dots2 - ksq[None, :]
    sv[:] = negs

    @pl.when(mi == NMB - 1)
    def _mask_tail():
        tail = lax.broadcasted_iota(jnp.int32, (QT, BM), 1)
        sv[:] = jnp.where(tail >= M - (NMB - 1) * BM, NEG, sv[:])

    done[0] = 0

    iw = lax.broadcasted_iota(jnp.int32, (QT, BM), 1)

    def body(r, _):
        @pl.when(done[0] == 0)
        def _round():
            s = sv[:]
            maxv = jnp.max(s, axis=-1, keepdims=True)
            rv = run_v[:]
            rmin = jnp.min(rv, axis=-1, keepdims=True)
            imp = maxv > rmin
            nimp = jnp.max(imp.astype(jnp.int32))
            done[0] = 1 - nimp

            @pl.when(nimp == 1)
            def _insert():
                p = jnp.min(jnp.where(s == maxv, iw, BM), axis=-1,
                            keepdims=True)
                sv[:] = jnp.where(iw == p, NEG, s)
                ri = run_i[:]
                ismin = rv == rmin
                mri = jnp.max(jnp.where(ismin, ri, -1), axis=-1,
                              keepdims=True)
                repl = ismin & (ri == mri) & imp
                run_v[:] = jnp.where(repl, maxv, rv)
                run_i[:] = jnp.where(repl, mi * BM + p, run_i[:])

        return 0

    lax.fori_loop(0, K, body, 0, unroll=False)

    @pl.when(mi == NMB - 1)
    def _out():
        idx_ref[:] = run_i[:]


def _topk_call(queries, memory_kv):
    return pl.pallas_call(
        _topk_body,
        grid=(NMB, Q // QT),
        in_specs=[
            pl.BlockSpec((QT, D), lambda mi, qi: (qi, 0)),
            pl.BlockSpec((BM, 2, D), lambda mi, qi: (mi, 0, 0)),
        ],
        out_specs=pl.BlockSpec((QT, K), lambda mi, qi: (qi, 0)),
        out_shape=jax.ShapeDtypeStruct((Q, K), jnp.int32),
        scratch_shapes=[
            pltpu.VMEM((Q // QT, QT, K), jnp.float32),
            pltpu.VMEM((Q // QT, QT, K), jnp.int32),
            pltpu.VMEM((QT, BM), jnp.float32),
            pltpu.SMEM((1,), jnp.int32),
        ],
    )(queries, memory_kv)


NW = 32
ROWS = Q * K
RPW = ROWS // NW
CH = 128
NCH = RPW // CH


def _gather_call(table, idx):
    mesh = plsc.VectorSubcoreMesh(core_axis_name="c", subcore_axis_name="s")

    @functools.partial(
        pl.kernel,
        mesh=mesh,
        out_type=jax.ShapeDtypeStruct((ROWS, 2 * D), jnp.float32),
        scratch_types=[
            pltpu.VMEM((RPW,), jnp.int32),
            pltpu.VMEM((CH, 2 * D), jnp.float32),
            pltpu.VMEM((CH, 2 * D), jnp.float32),
            pltpu.SemaphoreType.DMA,
            pltpu.SemaphoreType.DMA,
        ],
    )
    def k(table_hbm, idx_hbm, out_hbm, idx_v, rows0, rows1, sem0, sem1):
        wid = lax.axis_index("s") * 2 + lax.axis_index("c")
        rows = (rows0, rows1)
        sems = (sem0, sem1)
        pltpu.sync_copy(idx_hbm.at[pl.ds(wid * RPW, RPW)], idx_v)
        cps = [None, None]
        cps[0] = pltpu.async_copy(
            table_hbm.at[idx_v.at[pl.ds(0, CH)]], rows0, sem0)
        for c in range(NCH):
            nxt = c + 1
            if nxt < NCH:
                cps[nxt % 2] = pltpu.async_copy(
                    table_hbm.at[idx_v.at[pl.ds(nxt * CH, CH)]],
                    rows[nxt % 2], sems[nxt % 2])
            cps[c % 2].wait()
            pltpu.sync_copy(rows[c % 2],
                            out_hbm.at[pl.ds(wid * RPW + c * CH, CH)])

    return k(table, idx)


def _attn_body(topk_ref, q_ref, kv_ref, out_ref):
    q = q_ref[:]
    qn = q / (jnp.sqrt(jnp.sum(q * q, axis=-1, keepdims=True)) + 1e-12)
    rk = kv_ref[:, :, :D]
    rv = kv_ref[:, :, D:]
    scale = 1.0 / jnp.sqrt(jnp.float32(D))
    logits = jnp.sum(qn[:, None, :] * rk, axis=-1) * scale
    valid = lax.broadcasted_iota(jnp.int32, (QA, K), 1) < topk_ref[0]
    logits = jnp.where(valid, logits, NEG)
    m = jnp.max(logits, axis=-1, keepdims=True)
    e = jnp.exp(logits - m)
    w = e / jnp.sum(e, axis=-1, keepdims=True)
    out_ref[:] = jnp.sum(w[:, :, None] * rv, axis=1)


def _attn_call(topk, queries, kv):
    return pl.pallas_call(
        _attn_body,
        grid=(Q // QA,),
        in_specs=[
            pl.BlockSpec(memory_space=pltpu.SMEM),
            pl.BlockSpec((QA, D), lambda qi: (qi, 0)),
            pl.BlockSpec((QA, K, 2 * D), lambda qi: (qi, 0, 0)),
        ],
        out_specs=pl.BlockSpec((QA, D), lambda qi: (qi, 0)),
        out_shape=jax.ShapeDtypeStruct((Q, D), jnp.float32),
    )(topk, queries, kv)


def kernel(queries, memory_kv, topk):
    idx = _topk_call(queries, memory_kv)
    table = memory_kv.reshape(M, 2 * D)
    kv = _gather_call(table, idx.reshape(ROWS)).reshape(Q, K, 2 * D)
    topk_arr = jnp.asarray(topk, jnp.int32).reshape(1)
    return _attn_call(topk_arr, queries, kv)


# speedup vs baseline: 1.5118x; 1.2389x over previous
import functools

import jax
import jax.numpy as jnp
from jax import lax
from jax.experimental import pallas as pl
from jax.experimental.pallas import tpu as pltpu
from jax.experimental.pallas import tpu_sc as plsc

Q = 1024
D = 128
M = 100000
K = 32
BM = 1024
QT = 1024
QA = 256
NMB = (M + BM - 1) // BM

NEG = float("-inf")


def _topk_body(q_ref, kv_ref, idx_ref, run_vs, run_is, sv, done):
    mi = pl.program_id(0)
    qi = pl.program_id(1)
    run_v = run_vs.at[qi]
    run_i = run_is.at[qi]

    @pl.when(mi == 0)
    def _init():
        run_v[:] = jnp.full((QT, K), NEG, jnp.float32)
        run_i[:] = lax.broadcasted_iota(jnp.int32, (QT, K), 1)

    q = q_ref[:]
    qn = q / (jnp.sqrt(jnp.sum(q * q, axis=-1, keepdims=True)) + 1e-12)
    kb = kv_ref[:, 0, :]
    dots2 = lax.dot_general(qn + qn, kb, (((1,), (1,)), ((), ())),
                            preferred_element_type=jnp.float32)
    ksq = jnp.sum(kb * kb, axis=-1)
    negs = dots2 - ksq[None, :]
    sv[:] = negs

    @pl.when(mi == NMB - 1)
    def _mask_tail():
        tail = lax.broadcasted_iota(jnp.int32, (QT, BM), 1)
        sv[:] = jnp.where(tail >= M - (NMB - 1) * BM, NEG, sv[:])

    done[0] = 0

    iw = lax.broadcasted_iota(jnp.int32, (QT, BM), 1)

    def body(r, _):
        @pl.when(done[0] == 0)
        def _round():
            s = sv[:]
            maxv = jnp.max(s, axis=-1, keepdims=True)
            rv = run_v[:]
            rmin = jnp.min(rv, axis=-1, keepdims=True)
            imp = maxv > rmin
            nimp = jnp.max(imp.astype(jnp.int32))
            done[0] = 1 - nimp

            @pl.when(nimp == 1)
            def _insert():
                p = jnp.min(jnp.where(s == maxv, iw, BM), axis=-1,
                            keepdims=True)
                sv[:] = jnp.where(iw == p, NEG, s)
                ri = run_i[:]
                ismin = rv == rmin
                mri = jnp.max(jnp.where(ismin, ri, -1), axis=-1,
                              keepdims=True)
                repl = ismin & (ri == mri) & imp
                run_v[:] = jnp.where(repl, maxv, rv)
                run_i[:] = jnp.where(repl, mi * BM + p, run_i[:])

        return 0

    lax.fori_loop(0, K, body, 0, unroll=False)

    @pl.when(mi == NMB - 1)
    def _out():
        idx_ref[:] = run_i[:]


def _topk_call(queries, memory_kv):
    return pl.pallas_call(
        _topk_body,
        grid=(NMB, Q // QT),
        in_specs=[
            pl.BlockSpec((QT, D), lambda mi, qi: (qi, 0)),
            pl.BlockSpec((BM, 2, D), lambda mi, qi: (mi, 0, 0)),
        ],
        out_specs=pl.BlockSpec((QT, K), lambda mi, qi: (qi, 0)),
        out_shape=jax.ShapeDtypeStruct((Q, K), jnp.int32),
        scratch_shapes=[
            pltpu.VMEM((Q // QT, QT, K), jnp.float32),
            pltpu.VMEM((Q // QT, QT, K), jnp.int32),
            pltpu.VMEM((QT, BM), jnp.float32),
            pltpu.SMEM((1,), jnp.int32),
        ],
    )(queries, memory_kv)


NW = 32
ROWS = Q * K
RPW = ROWS // NW
CH = 128
NCH = RPW // CH


def _gather_call(table, idx):
    mesh = plsc.VectorSubcoreMesh(core_axis_name="c", subcore_axis_name="s")

    @functools.partial(
        pl.kernel,
        mesh=mesh,
        out_type=jax.ShapeDtypeStruct((ROWS, 2 * D), jnp.float32),
        scratch_types=[
            pltpu.VMEM((RPW,), jnp.int32),
            pltpu.VMEM((CH, 2 * D), jnp.float32),
            pltpu.VMEM((CH, 2 * D), jnp.float32),
            pltpu.SemaphoreType.DMA,
            pltpu.SemaphoreType.DMA,
        ],
    )
    def k(table_hbm, idx_hbm, out_hbm, idx_v, rows0, rows1, sem0, sem1):
        wid = lax.axis_index("s") * 2 + lax.axis_index("c")
        rows = (rows0, rows1)
        sems = (sem0, sem1)
        pltpu.sync_copy(idx_hbm.at[pl.ds(wid * RPW, RPW)], idx_v)
        cps = [None, None]
        cps[0] = pltpu.async_copy(
            table_hbm.at[idx_v.at[pl.ds(0, CH)]], rows0, sem0)
        for c in range(NCH):
            nxt = c + 1
            if nxt < NCH:
                cps[nxt % 2] = pltpu.async_copy(
                    table_hbm.at[idx_v.at[pl.ds(nxt * CH, CH)]],
                    rows[nxt % 2], sems[nxt % 2])
            cps[c % 2].wait()
            pltpu.sync_copy(rows[c % 2],
                            out_hbm.at[pl.ds(wid * RPW + c * CH, CH)])

    return k(table, idx)


def _attn_body(topk_ref, q_ref, kv_ref, out_ref):
    q = q_ref[:]
    qn = q / (jnp.sqrt(jnp.sum(q * q, axis=-1, keepdims=True)) + 1e-12)
    rk = kv_ref[:, :, :D]
    rv = kv_ref[:, :, D:]
    scale = 1.0 / jnp.sqrt(jnp.float32(D))
    logits = jnp.sum(qn[:, None, :] * rk, axis=-1) * scale
    valid = lax.broadcasted_iota(jnp.int32, (QA, K), 1) < topk_ref[0]
    logits = jnp.where(valid, logits, NEG)
    m = jnp.max(logits, axis=-1, keepdims=True)
    e = jnp.exp(logits - m)
    w = e / jnp.sum(e, axis=-1, keepdims=True)
    out_ref[:] = jnp.sum(w[:, :, None] * rv, axis=1)


def _attn_call(topk, queries, kv):
    return pl.pallas_call(
        _attn_body,
        grid=(Q // QA,),
        in_specs=[
            pl.BlockSpec(memory_space=pltpu.SMEM),
            pl.BlockSpec((QA, D), lambda qi: (qi, 0)),
            pl.BlockSpec((QA, K, 2 * D), lambda qi: (qi, 0, 0)),
        ],
        out_specs=pl.BlockSpec((QA, D), lambda qi: (qi, 0)),
        out_shape=jax.ShapeDtypeStruct((Q, D), jnp.float32),
    )(topk, queries, kv)


def kernel(queries, memory_kv, topk):
    idx = _topk_call(queries, memory_kv)
    table = memory_kv.reshape(M, 2 * D)
    kv = _gather_call(table, idx.reshape(ROWS)).reshape(Q, K, 2 * D)
    topk_arr = jnp.asarray(topk, jnp.int32).reshape(1)
    return _attn_call(topk_arr, queries, kv)
